# baseline (speedup 1.0000x reference)
import functools

import numpy as np

import jax
import jax.numpy as jnp
from jax.experimental import pallas as pl
from jax.experimental.pallas import tpu as pltpu

_LAYER_CHANNELS = ((3, 64), (64, 64), (64, 64), (64, 32), (32, 32),
                   (32, 32), (32, 16), (16, 16), (16, 16))
_NUM_LAYERS = len(_LAYER_CHANNELS)
_OUT_C = 16
_LANE = 128
_ROWL = 256
_SLOT = 64
_PX = 4
_NGRP = 12
_KDIM = _NGRP * _LANE


def _round_up(x, m):
    return ((x + m - 1) // m) * m


def _downsample_half(x_nhwc):
    n, h, w, c = x_nhwc.shape
    ho, wo = h // 2, w // 2
    x = x_nhwc[:, :2 * ho, :2 * wo, :].reshape(n, ho, 2, wo, 2, c)
    return x.mean(axis=(2, 4))


def _plan(scale_dims):
    pre = 24
    blocks = []
    cur = pre
    for (h, w) in scale_dims:
        wp = _round_up(w + 1, 32)
        gg = wp // _PX
        img_g = h * wp // _PX
        rows = _round_up(gg + img_g + gg, 8)
        blocks.append(dict(h=h, w=w, wp=wp, block_start=cur,
                           img_start=cur + gg, img_g=img_g, rows=rows))
        cur += rows
    m2 = cur + 16
    return blocks, m2


def _build_mask(blocks, m2):
    mask = np.zeros((m2, _ROWL), np.float32)
    for b in blocks:
        h, w, wp = b["h"], b["w"], b["wp"]
        p = np.arange(h * wp)
        valid = (p % wp < w).astype(np.float32)
        g0 = b["img_start"]
        for o in range(_PX):
            mask[g0:g0 + b["img_g"], o * _SLOT:(o + 1) * _SLOT] = (
                valid[o::_PX, None])
    return mask


def _build_wblk(w, cin, cout):
    z = jnp.zeros((_NGRP, 2, _SLOT, _PX, _SLOT), jnp.float32)
    for dy in (-1, 0, 1):
        for dx in (-1, 0, 1):
            for o in range(_PX):
                q = (o + dx) // 2
                slot = (o + dx) % 2
                t = 4 * (dy + 1) + (q + 1)
                z = z.at[t, slot, :cin, o, :cout].set(w[dy + 1, dx + 1])
    return z.reshape(_KDIM, _ROWL)


def _pyramid_kernel(chunks, split, img_ref, mask_ref, w_ref, b_ref, out_ref,
                    act_ref, actm_ref, actp_ref, slab_a, slab_b):
    m2 = img_ref.shape[0]
    _SPLIT = split
    act_ref[...] = img_ref[...].astype(jnp.float32)
    first = chunks[0][0]
    last = chunks[-1][0] + chunks[-1][1]
    slab_a[0:first, :] = jnp.zeros((first, _KDIM), jnp.float32)
    if _SPLIT < m2:
        slab_b[last - _SPLIT:m2 - _SPLIT, :] = jnp.zeros(
            (m2 - last, _KDIM), jnp.float32)
    else:
        slab_a[last:m2, :] = jnp.zeros((m2 - last, _KDIM), jnp.float32)
    actm_ref[0:8, :] = jnp.zeros((8, _ROWL), jnp.float32)
    actp_ref[m2 - 8:m2, :] = jnp.zeros((8, _ROWL), jnp.float32)

    def assemble(half_chunks, slab_ref, base):
        for (g0, rows, wp) in half_chunks:
            for dy in (-1, 0, 1):
                s = dy * (wp // _PX)
                for q in (-1, 0, 1, 2):
                    src, hf = ((actm_ref, 1), (act_ref, 0),
                               (act_ref, 1), (actp_ref, 0))[q + 1]
                    t = 4 * (dy + 1) + (q + 1)
                    slab_ref[g0 - base:g0 - base + rows,
                             t * _LANE:(t + 1) * _LANE] = (
                        src[g0 + s:g0 + s + rows,
                            hf * _LANE:(hf + 1) * _LANE])

    chunks_a, chunks_b = [], []
    for (g0, rows, wp) in chunks:
        if g0 < _SPLIT:
            ra = min(rows, _SPLIT - g0)
            chunks_a.append((g0, ra, wp))
            if rows > ra:
                chunks_b.append((g0 + ra, rows - ra, wp))
        else:
            chunks_b.append((g0, rows, wp))

    def layer_math(l, slab_ref, r0, r1):
        y = jnp.dot(slab_ref[...], w_ref[l],
                    preferred_element_type=jnp.float32)
        y = y + b_ref[l]
        y = jnp.maximum(y, 0.1 * y)
        return y * mask_ref[r0:r1, :]

    def layer(l):
        actp_ref[0:m2 - 8, :] = act_ref[1:m2 - 7, :]
        actm_ref[8:m2, :] = act_ref[7:m2 - 1, :]
        assemble(chunks_a, slab_a, 0)
        assemble(chunks_b, slab_b, _SPLIT)
        ya = layer_math(l, slab_a, 0, _SPLIT)
        yb = (layer_math(l, slab_b, _SPLIT, m2) if _SPLIT < m2 else None)
        return ya, yb

    for l in range(_NUM_LAYERS - 1):
        ya, yb = layer(l)
        act_ref[0:_SPLIT, :] = ya
        if yb is not None:
            act_ref[_SPLIT:m2, :] = yb
    def compact16(y):
        return jnp.concatenate(
            [y[:, o * _SLOT:o * _SLOT + _OUT_C] for o in range(_PX)], axis=1)

    ya, yb = layer(_NUM_LAYERS - 1)
    out_ref[0:_SPLIT, :] = compact16(ya).astype(jnp.bfloat16)
    if yb is not None:
        out_ref[_SPLIT:m2, :] = compact16(yb).astype(jnp.bfloat16)


@functools.partial(jax.jit, static_argnames=())
def _forward(params, img_nchw):
    img = jnp.transpose(img_nchw, (0, 2, 3, 1)).astype(jnp.float32)
    n = img.shape[0]

    pyr = [img]
    for _ in range(4):
        pyr.append(_downsample_half(pyr[-1]))
    scale_dims = tuple((int(p.shape[1]), int(p.shape[2])) for p in pyr)
    blocks, m2 = _plan(scale_dims)

    pieces = [jnp.zeros((n, blocks[0]["block_start"], _ROWL), jnp.bfloat16)]
    for p, b in zip(pyr, blocks):
        h, w, wp = b["h"], b["w"], b["wp"]
        q = jnp.pad(p.astype(jnp.bfloat16), ((0, 0), (0, 0), (0, wp - w), (0, 0)))
        q = q.reshape(n, h * wp // _PX, _PX, 3)
        q = jnp.pad(q, ((0, 0), (0, 0), (0, 0), (0, _SLOT - 3)))
        q = q.reshape(n, h * wp // _PX, _ROWL)
        gg = b["img_start"] - b["block_start"]
        tail = b["rows"] - gg - b["img_g"]
        pieces.append(jnp.zeros((n, gg, _ROWL), jnp.bfloat16))
        pieces.append(q)
        pieces.append(jnp.zeros((n, tail, _ROWL), jnp.bfloat16))
    pieces.append(jnp.zeros((n, m2 - blocks[-1]["block_start"] - blocks[-1]["rows"],
                             _ROWL), jnp.bfloat16))
    img_packed = jnp.concatenate(pieces, axis=1)

    mask = jnp.asarray(_build_mask(blocks, m2))

    w_stack = jnp.stack([
        _build_wblk(w, cin, cout)
        for (w, _), (cin, cout) in zip(params, _LAYER_CHANNELS)
    ])
    b_stack = jnp.stack([
        jnp.pad(b, (0, _SLOT - b.shape[0]))
        for (_, b) in params])
    b_stack = jnp.concatenate([b_stack] * _PX, axis=-1).reshape(
        _NUM_LAYERS, 1, _ROWL)

    chunks = tuple((b["block_start"], b["rows"], b["wp"]) for b in blocks)
    split = m2
    kfn = functools.partial(_pyramid_kernel, chunks, split)

    out = pl.pallas_call(
        kfn,
        grid=(n,),
        in_specs=[
            pl.BlockSpec((None, m2, _ROWL), lambda i: (i, 0, 0)),
            pl.BlockSpec((m2, _ROWL), lambda i: (0, 0)),
            pl.BlockSpec((_NUM_LAYERS, _KDIM, _ROWL), lambda i: (0, 0, 0)),
            pl.BlockSpec((_NUM_LAYERS, 1, _ROWL), lambda i: (0, 0, 0)),
        ],
        out_specs=pl.BlockSpec((None, m2, _SLOT), lambda i: (i, 0, 0)),
        out_shape=jax.ShapeDtypeStruct((n, m2, _SLOT), jnp.bfloat16),
        scratch_shapes=[
            pltpu.VMEM((m2, _ROWL), jnp.float32),
            pltpu.VMEM((m2, _ROWL), jnp.float32),
            pltpu.VMEM((m2, _ROWL), jnp.float32),
            pltpu.VMEM((split, _KDIM), jnp.float32),
            pltpu.VMEM((max(m2 - split, 8), _KDIM), jnp.float32),
        ],
        compiler_params=pltpu.CompilerParams(
            dimension_semantics=("parallel",),
            vmem_limit_bytes=60 * 1024 * 1024),
    )(img_packed, mask, w_stack, b_stack)

    feats = []
    for b in blocks:
        h, w, wp = b["h"], b["w"], b["wp"]
        f = out[:, b["img_start"]:b["img_start"] + b["img_g"], :]
        f = f.reshape(n, h * wp // _PX, _PX, _OUT_C).reshape(n, h, wp, _OUT_C)
        f = f[:, :, :w, :].astype(jnp.float32)
        feats.append(jnp.transpose(f, (0, 3, 1, 2)))
    return feats


def kernel(w0, b0, w1, b1, w2, b2, w3, b3, w4, b4,
           w5, b5, w6, b6, w7, b7, w8, b8, img):
    params = [(w0, b0), (w1, b1), (w2, b2), (w3, b3), (w4, b4),
              (w5, b5), (w6, b6), (w7, b7), (w8, b8)]
    return _forward(params, img)


# speedup vs baseline: 1.4388x; 1.4388x over previous
import functools

import numpy as np

import jax
import jax.numpy as jnp
from jax.experimental import pallas as pl
from jax.experimental.pallas import tpu as pltpu

_LAYER_CHANNELS = ((3, 64), (64, 64), (64, 64), (64, 32), (32, 32),
                   (32, 32), (32, 16), (16, 16), (16, 16))
_NUM_LAYERS = len(_LAYER_CHANNELS)
_OUT_C = 16
_LANE = 128
_ROWL = 256
_SLOT = 64
_PX = 4
_NGRP = 12
_KDIM = _NGRP * _LANE


def _round_up(x, m):
    return ((x + m - 1) // m) * m


def _downsample_half(x_nhwc):
    n, h, w, c = x_nhwc.shape
    ho, wo = h // 2, w // 2
    x = x_nhwc[:, :2 * ho, :2 * wo, :].reshape(n, ho, 2, wo, 2, c)
    return x.mean(axis=(2, 4))


def _plan(scale_dims):
    pre = 24
    blocks = []
    cur = pre
    for (h, w) in scale_dims:
        wp = _round_up(w + 1, 32)
        gg = wp // _PX
        img_g = h * wp // _PX
        rows = _round_up(gg + img_g + gg, 8)
        blocks.append(dict(h=h, w=w, wp=wp, block_start=cur,
                           img_start=cur + gg, img_g=img_g, rows=rows))
        cur += rows
    m2 = cur + 16
    return blocks, m2


def _build_mask(blocks, m2):
    mask = np.zeros((m2, _ROWL), np.float32)
    for b in blocks:
        h, w, wp = b["h"], b["w"], b["wp"]
        p = np.arange(h * wp)
        valid = (p % wp < w).astype(np.float32)
        g0 = b["img_start"]
        for o in range(_PX):
            mask[g0:g0 + b["img_g"], o * _SLOT:(o + 1) * _SLOT] = (
                valid[o::_PX, None])
    return mask


def _build_wblk(w, cin, cout):
    z = jnp.zeros((_NGRP, 2, _SLOT, _PX, _SLOT), jnp.float32)
    for dy in (-1, 0, 1):
        for dx in (-1, 0, 1):
            for o in range(_PX):
                q = (o + dx) // 2
                slot = (o + dx) % 2
                t = 4 * (dy + 1) + (q + 1)
                z = z.at[t, slot, :cin, o, :cout].set(w[dy + 1, dx + 1])
    return z.reshape(_KDIM, _ROWL)


def _pyramid_kernel(chunks, split, img_ref, mask_ref, w_ref, b_ref, out_ref,
                    act_ref, actm_ref, actp_ref, slab_a, slab_b):
    m2 = img_ref.shape[0]
    _SPLIT = split
    act_ref[...] = img_ref[...].astype(jnp.float32)
    first = chunks[0][0]
    last = chunks[-1][0] + chunks[-1][1]
    slab_a[0:first, :] = jnp.zeros((first, _KDIM), jnp.float32)
    slab_b[last - _SPLIT:m2 - _SPLIT, :] = jnp.zeros(
        (m2 - last, _KDIM), jnp.float32)
    actm_ref[0:8, :] = jnp.zeros((8, _ROWL), jnp.float32)
    actp_ref[m2 - 8:m2, :] = jnp.zeros((8, _ROWL), jnp.float32)

    def assemble(half_chunks, slab_ref, base):
        for (g0, rows, wp) in half_chunks:
            for dy in (-1, 0, 1):
                s = dy * (wp // _PX)
                for q in (-1, 0, 1, 2):
                    src, hf = ((actm_ref, 1), (act_ref, 0),
                               (act_ref, 1), (actp_ref, 0))[q + 1]
                    t = 4 * (dy + 1) + (q + 1)
                    slab_ref[g0 - base:g0 - base + rows,
                             t * _LANE:(t + 1) * _LANE] = (
                        src[g0 + s:g0 + s + rows,
                            hf * _LANE:(hf + 1) * _LANE])

    chunks_a, chunks_b = [], []
    for (g0, rows, wp) in chunks:
        if g0 < _SPLIT:
            ra = min(rows, _SPLIT - g0)
            chunks_a.append((g0, ra, wp))
            if rows > ra:
                chunks_b.append((g0 + ra, rows - ra, wp))
        else:
            chunks_b.append((g0, rows, wp))

    def layer_math(l, slab_ref, r0, r1):
        y = jnp.dot(slab_ref[...], w_ref[l],
                    preferred_element_type=jnp.float32)
        y = y + b_ref[l]
        y = jnp.maximum(y, 0.1 * y)
        return y * mask_ref[r0:r1, :]

    def layer(l):
        actp_ref[0:m2 - 8, :] = act_ref[1:m2 - 7, :]
        actm_ref[8:m2, :] = act_ref[7:m2 - 1, :]
        assemble(chunks_a, slab_a, 0)
        assemble(chunks_b, slab_b, _SPLIT)
        ya = layer_math(l, slab_a, 0, _SPLIT)
        yb = layer_math(l, slab_b, _SPLIT, m2)
        return ya, yb

    for l in range(_NUM_LAYERS - 1):
        ya, yb = layer(l)
        act_ref[0:_SPLIT, :] = ya
        act_ref[_SPLIT:m2, :] = yb
    def compact16(y):
        return jnp.concatenate(
            [y[:, o * _SLOT:o * _SLOT + _OUT_C] for o in range(_PX)], axis=1)

    ya, yb = layer(_NUM_LAYERS - 1)
    out_ref[0:_SPLIT, :] = compact16(ya).astype(jnp.bfloat16)
    out_ref[_SPLIT:m2, :] = compact16(yb).astype(jnp.bfloat16)


@functools.partial(jax.jit, static_argnames=())
def _forward(params, img_nchw):
    img = jnp.transpose(img_nchw, (0, 2, 3, 1)).astype(jnp.float32)
    n = img.shape[0]

    pyr = [img]
    for _ in range(4):
        pyr.append(_downsample_half(pyr[-1]))
    scale_dims = tuple((int(p.shape[1]), int(p.shape[2])) for p in pyr)
    blocks, m2 = _plan(scale_dims)

    pieces = [jnp.zeros((n, blocks[0]["block_start"], _ROWL), jnp.bfloat16)]
    for p, b in zip(pyr, blocks):
        h, w, wp = b["h"], b["w"], b["wp"]
        q = jnp.pad(p.astype(jnp.bfloat16), ((0, 0), (0, 0), (0, wp - w), (0, 0)))
        q = q.reshape(n, h * wp // _PX, _PX, 3)
        q = jnp.pad(q, ((0, 0), (0, 0), (0, 0), (0, _SLOT - 3)))
        q = q.reshape(n, h * wp // _PX, _ROWL)
        gg = b["img_start"] - b["block_start"]
        tail = b["rows"] - gg - b["img_g"]
        pieces.append(jnp.zeros((n, gg, _ROWL), jnp.bfloat16))
        pieces.append(q)
        pieces.append(jnp.zeros((n, tail, _ROWL), jnp.bfloat16))
    pieces.append(jnp.zeros((n, m2 - blocks[-1]["block_start"] - blocks[-1]["rows"],
                             _ROWL), jnp.bfloat16))
    img_packed = jnp.concatenate(pieces, axis=1)

    mask = jnp.asarray(_build_mask(blocks, m2))

    w_stack = jnp.stack([
        _build_wblk(w, cin, cout)
        for (w, _), (cin, cout) in zip(params, _LAYER_CHANNELS)
    ])
    b_stack = jnp.stack([
        jnp.pad(b, (0, _SLOT - b.shape[0]))
        for (_, b) in params])
    b_stack = jnp.concatenate([b_stack] * _PX, axis=-1).reshape(
        _NUM_LAYERS, 1, _ROWL)

    chunks = tuple((b["block_start"], b["rows"], b["wp"]) for b in blocks)
    split = _round_up(m2 // 2, 32)
    kfn = functools.partial(_pyramid_kernel, chunks, split)

    out = pl.pallas_call(
        kfn,
        grid=(n,),
        in_specs=[
            pl.BlockSpec((None, m2, _ROWL), lambda i: (i, 0, 0)),
            pl.BlockSpec((m2, _ROWL), lambda i: (0, 0)),
            pl.BlockSpec((_NUM_LAYERS, _KDIM, _ROWL), lambda i: (0, 0, 0)),
            pl.BlockSpec((_NUM_LAYERS, 1, _ROWL), lambda i: (0, 0, 0)),
        ],
        out_specs=pl.BlockSpec((None, m2, _SLOT), lambda i: (i, 0, 0)),
        out_shape=jax.ShapeDtypeStruct((n, m2, _SLOT), jnp.bfloat16),
        scratch_shapes=[
            pltpu.VMEM((m2, _ROWL), jnp.float32),
            pltpu.VMEM((m2, _ROWL), jnp.float32),
            pltpu.VMEM((m2, _ROWL), jnp.float32),
            pltpu.VMEM((split, _KDIM), jnp.float32),
            pltpu.VMEM((m2 - split, _KDIM), jnp.float32),
        ],
        compiler_params=pltpu.CompilerParams(
            dimension_semantics=("parallel",),
            vmem_limit_bytes=60 * 1024 * 1024),
    )(img_packed, mask, w_stack, b_stack)

    feats = []
    for b in blocks:
        h, w, wp = b["h"], b["w"], b["wp"]
        f = out[:, b["img_start"]:b["img_start"] + b["img_g"], :]
        f = f.reshape(n, h * wp // _PX, _PX, _OUT_C).reshape(n, h, wp, _OUT_C)
        f = f[:, :, :w, :].astype(jnp.float32)
        feats.append(jnp.transpose(f, (0, 3, 1, 2)))
    return feats


def kernel(w0, b0, w1, b1, w2, b2, w3, b3, w4, b4,
           w5, b5, w6, b6, w7, b7, w8, b8, img):
    params = [(w0, b0), (w1, b1), (w2, b2), (w3, b3), (w4, b4),
              (w5, b5), (w6, b6), (w7, b7), (w8, b8)]
    return _forward(params, img)
